# baseline (speedup 1.0000x reference)
import functools

import jax
import jax.numpy as jnp
from jax import lax
from jax.experimental import pallas as pl
from jax.experimental.pallas import tpu as pltpu
from jax.experimental.pallas import tpu_sc as plsc

N = 100000
E = 6400000
G = 64
NC = 2
NS = 16
N_PAD = 100096
RPT = N_PAD // NS
K = 4000
KS = 4000
KW = 800
BN = 5000
NB = N // BN

f32 = jnp.float32


def _chunks(total, cap):
    out, off = [], 0
    while off < total:
        ln = min(cap, total - off)
        assert ln % 8 == 0 and off % 8 == 0
        out.append((off, ln))
        off += ln
    return out


def _zero_accum(zz, accum, bounce, s, cap):
    for boff, blen in _chunks(RPT, cap):
        off = s * RPT + boff
        pltpu.sync_copy(zz.at[pl.ds(off, blen)], bounce.at[pl.ds(0, blen)])
        pltpu.sync_copy(bounce.at[pl.ds(0, blen)], accum.at[pl.ds(off, blen)])


def _copy_out(accum, bounce, out_a, out_b, c, s, cap):
    def emit(o):
        for boff, blen in _chunks(RPT, cap):
            off = s * RPT + boff
            pltpu.sync_copy(accum.at[pl.ds(off, blen)],
                            bounce.at[pl.ds(0, blen)])
            pltpu.sync_copy(bounce.at[pl.ds(0, blen)], o.at[pl.ds(off, blen)])

    @pl.when(c == 0)
    def _():
        emit(out_a)

    @pl.when(c == 1)
    def _():
        emit(out_b)


@functools.cache
def _get_sc_deg():
    mesh = plsc.VectorSubcoreMesh(core_axis_name="c", subcore_axis_name="s")
    sds = jax.ShapeDtypeStruct((N_PAD, 8), f32)

    @functools.partial(
        pl.kernel,
        out_type=(sds, sds),
        mesh=mesh,
        compiler_params=pltpu.CompilerParams(use_tc_tiling_on_sc=False),
        scratch_types=[
            pltpu.VMEM_SHARED((N_PAD, 8), f32),
            pltpu.VMEM((K,), jnp.int32),
            pltpu.VMEM((K,), jnp.int32),
            pltpu.VMEM((K, 8), f32),
            pltpu.SemaphoreType.DMA,
            pltpu.SemaphoreType.DMA,
        ],
    )
    def _sc_deg(dst, ones_hbm, zz, out_a, out_b, accum, dstb0, dstb1, onesb,
                semi0, semi1):
        c = lax.axis_index("c")
        s = lax.axis_index("s")
        _zero_accum(zz, accum, onesb, s, K)
        pltpu.sync_copy(ones_hbm, onesb)
        plsc.subcore_barrier()
        epw = E // (NC * NS)
        nit = epw // K
        base0 = c * (E // 2) + s * epw
        bufs = ((dstb0, semi0), (dstb1, semi1))
        pltpu.async_copy(dst.at[pl.ds(base0, K)], dstb0, semi0)
        pltpu.async_copy(dst.at[pl.ds(base0 + K, K)], dstb1, semi1)

        def it(jj, carry):
            for b in range(2):
                j = 2 * jj + b
                dstb, semi = bufs[b]
                pltpu.make_async_copy(dst.at[pl.ds(0, K)], dstb, semi).wait()
                pltpu.sync_copy(onesb, accum.at[dstb], add=True)

                @pl.when(j + 2 < nit)
                def _():
                    pltpu.async_copy(dst.at[pl.ds(base0 + (j + 2) * K, K)],
                                     dstb, semi)
            return carry

        lax.fori_loop(0, nit // 2, it, 0)
        plsc.subcore_barrier()
        _copy_out(accum, onesb, out_a, out_b, c, s, K)

    return _sc_deg


@functools.cache
def _make_sc_scatter(F, Kc, split):
    mesh = plsc.VectorSubcoreMesh(core_axis_name="c", subcore_axis_name="s")
    sds = jax.ShapeDtypeStruct((N_PAD, F), f32)

    @functools.partial(
        pl.kernel,
        out_type=(sds, sds),
        mesh=mesh,
        compiler_params=pltpu.CompilerParams(use_tc_tiling_on_sc=False),
        scratch_types=[
            pltpu.VMEM_SHARED((N_PAD, F), f32),
            pltpu.VMEM((Kc,), jnp.int32),
            pltpu.VMEM((Kc,), jnp.int32),
            pltpu.VMEM((Kc, F), f32),
            pltpu.VMEM((Kc,), jnp.int32),
            pltpu.VMEM((Kc,), jnp.int32),
            pltpu.VMEM((Kc, F), f32),
            pltpu.SemaphoreType.DMA,
            pltpu.SemaphoreType.DMA,
            pltpu.SemaphoreType.DMA,
            pltpu.SemaphoreType.DMA,
        ],
    )
    def body(g, src, dst, zz, out_a, out_b, accum, srcb0, dstb0, rows0,
             srcb1, dstb1, rows1, semi0, semg0, semi1, semg1):
        c = lax.axis_index("c")
        s = lax.axis_index("s")
        _zero_accum(zz, accum, rows0, s, Kc)
        plsc.subcore_barrier()
        if split:
            epw = E // (NC * NS)
            base0 = c * (E // 2) + s * epw
        else:
            epw = E // NS
            base0 = s * epw
        sbase = base0
        nit = epw // Kc
        coff = jnp.int32(0) if split else c * jnp.int32(N)

        def adjust(srcb):
            if split:
                return

            def fix(i, carry):
                v = srcb[pl.ds(i * 16, 16)]
                srcb[pl.ds(i * 16, 16)] = v + coff
                return carry

            lax.fori_loop(0, Kc // 16, fix, 0)
        bufs = ((srcb0, dstb0, rows0, semi0, semg0),
                (srcb1, dstb1, rows1, semi1, semg1))
        pltpu.async_copy(src.at[pl.ds(sbase, Kc)], srcb0, semi0)
        pltpu.async_copy(dst.at[pl.ds(base0, Kc)], dstb0, semi0)
        pltpu.make_async_copy(src.at[pl.ds(0, Kc)], srcb0, semi0).wait()
        adjust(srcb0)
        pltpu.async_copy(g.at[srcb0], rows0, semg0)
        pltpu.async_copy(src.at[pl.ds(sbase + Kc, Kc)], srcb1, semi1)
        pltpu.async_copy(dst.at[pl.ds(base0 + Kc, Kc)], dstb1, semi1)

        def it(jj, carry):
            for b in range(2):
                j = 2 * jj + b
                srcb, dstb, rows, semi, semg = bufs[b]
                srcb_n, dstb_n, rows_n, semi_n, semg_n = bufs[1 - b]
                pltpu.make_async_copy(g.at[srcb], rows, semg).wait()

                @pl.when(j + 1 < nit)
                def _():
                    pltpu.make_async_copy(src.at[pl.ds(0, Kc)], srcb_n,
                                          semi_n).wait()
                    adjust(srcb_n)
                    pltpu.async_copy(g.at[srcb_n], rows_n, semg_n)

                pltpu.make_async_copy(dst.at[pl.ds(0, Kc)], dstb, semi).wait()
                pltpu.sync_copy(rows, accum.at[dstb], add=True)

                @pl.when(j + 2 < nit)
                def _():
                    o2 = (j + 2) * Kc
                    pltpu.async_copy(src.at[pl.ds(sbase + o2, Kc)], srcb, semi)
                    pltpu.async_copy(dst.at[pl.ds(base0 + o2, Kc)], dstb, semi)
            return carry

        lax.fori_loop(0, nit // 2, it, 0)
        plsc.subcore_barrier()
        _copy_out(accum, rows0, out_a, out_b, c, s, Kc)

    return body


def _row(F):
    return pl.BlockSpec((BN, F), lambda i: (i, 0))


def _tc1(x, dega, degb):
    def body(xr, dar, dbr, dinvr, g1r):
        dinv = lax.rsqrt(1.0 + dar[...][:, :1] + dbr[...][:, :1])
        dinvr[...] = dinv
        g1r[...] = jnp.concatenate(
            [xr[...], jnp.zeros((BN, 5), f32)], axis=1) * dinv

    return pl.pallas_call(
        body,
        grid=(NB,),
        in_specs=[_row(3), _row(8), _row(8)],
        out_specs=[_row(1), _row(8)],
        out_shape=[jax.ShapeDtypeStruct((N, 1), f32),
                   jax.ShapeDtypeStruct((N, 8), f32)],
    )(x, dega, degb)


def _tc2(s1a, s1b, g1, dinv, W1p, b1):
    def body(s1ar, s1br, g1r, dr, wr, br, g2r):
        h = pl.program_id(0)
        d = dr[...]
        p1 = d * (s1ar[...] + s1br[...] + g1r[...])
        h1 = jnp.maximum(
            jnp.dot(p1, wr[...], preferred_element_type=f32) + br[...], 0.0)
        g2 = h1 * d
        g2r[...] = jnp.where(h == 0, g2[:, :16], g2[:, 16:])

    return pl.pallas_call(
        body,
        grid=(2, NB),
        in_specs=[pl.BlockSpec((BN, 8), lambda h, i: (i, 0)),
                  pl.BlockSpec((BN, 8), lambda h, i: (i, 0)),
                  pl.BlockSpec((BN, 8), lambda h, i: (i, 0)),
                  pl.BlockSpec((BN, 1), lambda h, i: (i, 0)),
                  pl.BlockSpec((8, 32), lambda h, i: (0, 0)),
                  pl.BlockSpec((1, 32), lambda h, i: (0, 0))],
        out_specs=pl.BlockSpec((BN, 16), lambda h, i: (h * NB + i, 0)),
        out_shape=jax.ShapeDtypeStruct((2 * N, 16), f32),
    )(s1a, s1b, g1, dinv, W1p, b1)


def _tc3(s2a, s2b, g2, dinv, W2, b2, W3):
    def body(s2ar, s2br, g2ar, g2br, dr, w2r, b2r, w3r, g3r):
        d = dr[...]
        p2 = jnp.concatenate(
            [d * (s2ar[...] + g2ar[...]), d * (s2br[...] + g2br[...])], axis=1)
        h2 = jnp.maximum(
            jnp.dot(p2, w2r[...], preferred_element_type=f32) + b2r[...], 0.0)
        t3 = jnp.dot(h2, w3r[...], preferred_element_type=f32) * d
        g3r[...] = jnp.concatenate([t3, jnp.zeros((BN, 6), f32)], axis=1)

    return pl.pallas_call(
        body,
        grid=(NB,),
        in_specs=[_row(16), _row(16),
                  pl.BlockSpec((BN, 16), lambda i: (i, 0)),
                  pl.BlockSpec((BN, 16), lambda i: (NB + i, 0)),
                  _row(1),
                  pl.BlockSpec((32, 64), lambda i: (0, 0)),
                  pl.BlockSpec((1, 64), lambda i: (0, 0)),
                  pl.BlockSpec((64, 2), lambda i: (0, 0))],
        out_specs=_row(8),
        out_shape=jax.ShapeDtypeStruct((N, 8), f32),
    )(s2a, s2b, g2, g2, dinv, W2, b2, W3)


def _tc4(s3a, s3b, g3, dinv, batT, b3):
    def body(s3ar, s3br, g3r, dr, batr, b3r, outr, sums, cnt):
        i = pl.program_id(0)
        q = dr[...] * (s3ar[...][:, :2] + s3br[...][:, :2]
                       + g3r[...][:, :2])
        gid = lax.broadcasted_iota(jnp.int32, (G, BN), 0)
        maskT = (batr[0] == gid).astype(f32)

        @pl.when(i == 0)
        def _():
            sums[...] = jnp.zeros_like(sums)
            cnt[...] = jnp.zeros_like(cnt)

        sums[...] += jnp.dot(maskT, q, preferred_element_type=f32)
        cnt[...] += jnp.dot(maskT, jnp.ones((BN, 2), f32),
                            preferred_element_type=f32)

        @pl.when(i == NB - 1)
        def _():
            c = cnt[...]
            outr[...] = (sums[...] / jnp.maximum(c, 1.0)
                         + jnp.where(c > 0, b3r[...], 0.0))

    return pl.pallas_call(
        body,
        grid=(NB,),
        in_specs=[_row(8), _row(8), _row(8), _row(1),
                  pl.BlockSpec((1, 1, BN), lambda i: (i, 0, 0)),
                  pl.BlockSpec((1, 2), lambda i: (0, 0))],
        out_specs=pl.BlockSpec((G, 2), lambda i: (0, 0)),
        out_shape=jax.ShapeDtypeStruct((G, 2), f32),
        scratch_shapes=[pltpu.VMEM((G, 2), f32), pltpu.VMEM((G, 2), f32)],
    )(s3a, s3b, g3, dinv, batT, b3)


def kernel(x, edge_index, batch, W1, b1, W2, b2, W3, b3):
    src = edge_index[0]
    dst = edge_index[1]
    W1p = jnp.pad(W1, ((0, 5), (0, 0)))

    dega, degb = _get_sc_deg()(dst, jnp.ones((K, 8), f32),
                               jnp.zeros((N_PAD, 8), f32))
    dinv, g1 = _tc1(x, dega, degb)

    s1a, s1b = _make_sc_scatter(8, KS, True)(
        g1, src, dst, jnp.zeros((N_PAD, 8), f32))
    g2 = _tc2(s1a, s1b, g1, dinv, W1p, b1.reshape(1, 32))

    s2a, s2b = _make_sc_scatter(16, KW, False)(
        g2, src, dst, jnp.zeros((N_PAD, 16), f32))
    g3 = _tc3(s2a, s2b, g2, dinv, W2, b2.reshape(1, 64), W3)

    s3a, s3b = _make_sc_scatter(8, KS, True)(
        g3, src, dst, jnp.zeros((N_PAD, 8), f32))
    return _tc4(s3a, s3b, g3, dinv,
                batch.reshape(NB, 1, BN), b3.reshape(1, 2))


# speedup vs baseline: 1.0352x; 1.0352x over previous
import functools

import jax
import jax.numpy as jnp
from jax import lax
from jax.experimental import pallas as pl
from jax.experimental.pallas import tpu as pltpu
from jax.experimental.pallas import tpu_sc as plsc

N = 100000
E = 6400000
G = 64
NC = 2
NS = 16
N_PAD = 100096
RPT = N_PAD // NS
K = 4000
KS = 4000
KW = 800
BN = 5000
NB = N // BN

f32 = jnp.float32


def _chunks(total, cap):
    out, off = [], 0
    while off < total:
        ln = min(cap, total - off)
        assert ln % 8 == 0 and off % 8 == 0
        out.append((off, ln))
        off += ln
    return out


def _zero_accum(zz, accum, bounce, s, cap):
    for boff, blen in _chunks(RPT, cap):
        off = s * RPT + boff
        pltpu.sync_copy(zz.at[pl.ds(off, blen)], bounce.at[pl.ds(0, blen)])
        pltpu.sync_copy(bounce.at[pl.ds(0, blen)], accum.at[pl.ds(off, blen)])


def _copy_out(accum, bounce, out_a, out_b, c, s, cap):
    def emit(o):
        for boff, blen in _chunks(RPT, cap):
            off = s * RPT + boff
            pltpu.sync_copy(accum.at[pl.ds(off, blen)],
                            bounce.at[pl.ds(0, blen)])
            pltpu.sync_copy(bounce.at[pl.ds(0, blen)], o.at[pl.ds(off, blen)])

    @pl.when(c == 0)
    def _():
        emit(out_a)

    @pl.when(c == 1)
    def _():
        emit(out_b)


@functools.cache
def _get_sc_deg():
    mesh = plsc.VectorSubcoreMesh(core_axis_name="c", subcore_axis_name="s")
    sds = jax.ShapeDtypeStruct((N_PAD, 8), f32)

    @functools.partial(
        pl.kernel,
        out_type=(sds, sds),
        mesh=mesh,
        compiler_params=pltpu.CompilerParams(use_tc_tiling_on_sc=False),
        scratch_types=[
            pltpu.VMEM_SHARED((N_PAD, 8), f32),
            pltpu.VMEM((K,), jnp.int32),
            pltpu.VMEM((K,), jnp.int32),
            pltpu.VMEM((K, 8), f32),
            pltpu.SemaphoreType.DMA,
            pltpu.SemaphoreType.DMA,
        ],
    )
    def _sc_deg(dst, ones_hbm, zz, out_a, out_b, accum, dstb0, dstb1, onesb,
                semi0, semi1):
        c = lax.axis_index("c")
        s = lax.axis_index("s")
        _zero_accum(zz, accum, onesb, s, K)
        pltpu.sync_copy(ones_hbm, onesb)
        plsc.subcore_barrier()
        epw = E // (NC * NS)
        nit = epw // K
        base0 = c * (E // 2) + s * epw
        bufs = ((dstb0, semi0), (dstb1, semi1))
        pltpu.async_copy(dst.at[pl.ds(base0, K)], dstb0, semi0)
        pltpu.async_copy(dst.at[pl.ds(base0 + K, K)], dstb1, semi1)

        def it(jj, carry):
            for b in range(2):
                j = 2 * jj + b
                dstb, semi = bufs[b]
                pltpu.make_async_copy(dst.at[pl.ds(0, K)], dstb, semi).wait()
                pltpu.sync_copy(onesb, accum.at[dstb], add=True)

                @pl.when(j + 2 < nit)
                def _():
                    pltpu.async_copy(dst.at[pl.ds(base0 + (j + 2) * K, K)],
                                     dstb, semi)
            return carry

        lax.fori_loop(0, nit // 2, it, 0)
        plsc.subcore_barrier()
        _copy_out(accum, onesb, out_a, out_b, c, s, K)

    return _sc_deg


@functools.cache
def _make_sc_scatter(F, Kc, split):
    mesh = plsc.VectorSubcoreMesh(core_axis_name="c", subcore_axis_name="s")
    sds = jax.ShapeDtypeStruct((N_PAD, F), f32)

    @functools.partial(
        pl.kernel,
        out_type=(sds, sds),
        mesh=mesh,
        compiler_params=pltpu.CompilerParams(use_tc_tiling_on_sc=False),
        scratch_types=[
            pltpu.VMEM_SHARED((N_PAD, F), f32),
            pltpu.VMEM((Kc,), jnp.int32),
            pltpu.VMEM((Kc,), jnp.int32),
            pltpu.VMEM((Kc, F), f32),
            pltpu.VMEM((Kc,), jnp.int32),
            pltpu.VMEM((Kc,), jnp.int32),
            pltpu.VMEM((Kc, F), f32),
            pltpu.SemaphoreType.DMA,
            pltpu.SemaphoreType.DMA,
            pltpu.SemaphoreType.DMA,
            pltpu.SemaphoreType.DMA,
        ],
    )
    def body(g, src, dst, zz, out_a, out_b, accum, srcb0, dstb0, rows0,
             srcb1, dstb1, rows1, semi0, semg0, semi1, semg1):
        c = lax.axis_index("c")
        s = lax.axis_index("s")
        _zero_accum(zz, accum, rows0, s, Kc)
        plsc.subcore_barrier()
        if split:
            epw = E // (NC * NS)
            base0 = c * (E // 2) + s * epw
        else:
            epw = E // NS
            base0 = s * epw
        if not split:
            sbase = c * E + base0
        else:
            sbase = base0
        nit = epw // Kc
        bufs = ((srcb0, dstb0, rows0, semi0, semg0),
                (srcb1, dstb1, rows1, semi1, semg1))
        pltpu.async_copy(src.at[pl.ds(sbase, Kc)], srcb0, semi0)
        pltpu.async_copy(dst.at[pl.ds(base0, Kc)], dstb0, semi0)
        pltpu.make_async_copy(src.at[pl.ds(0, Kc)], srcb0, semi0).wait()
        pltpu.async_copy(g.at[srcb0], rows0, semg0)
        pltpu.async_copy(src.at[pl.ds(sbase + Kc, Kc)], srcb1, semi1)
        pltpu.async_copy(dst.at[pl.ds(base0 + Kc, Kc)], dstb1, semi1)

        def it(jj, carry):
            for b in range(2):
                j = 2 * jj + b
                srcb, dstb, rows, semi, semg = bufs[b]
                srcb_n, dstb_n, rows_n, semi_n, semg_n = bufs[1 - b]
                pltpu.make_async_copy(g.at[srcb], rows, semg).wait()

                @pl.when(j + 1 < nit)
                def _():
                    pltpu.make_async_copy(src.at[pl.ds(0, Kc)], srcb_n,
                                          semi_n).wait()
                    pltpu.async_copy(g.at[srcb_n], rows_n, semg_n)

                pltpu.make_async_copy(dst.at[pl.ds(0, Kc)], dstb, semi).wait()
                pltpu.sync_copy(rows, accum.at[dstb], add=True)

                @pl.when(j + 2 < nit)
                def _():
                    o2 = (j + 2) * Kc
                    pltpu.async_copy(src.at[pl.ds(sbase + o2, Kc)], srcb, semi)
                    pltpu.async_copy(dst.at[pl.ds(base0 + o2, Kc)], dstb, semi)
            return carry

        lax.fori_loop(0, nit // 2, it, 0)
        plsc.subcore_barrier()
        _copy_out(accum, rows0, out_a, out_b, c, s, Kc)

    return body


def _row(F):
    return pl.BlockSpec((BN, F), lambda i: (i, 0))


def _tc1(x, dega, degb):
    def body(xr, dar, dbr, dinvr, g1r):
        dinv = lax.rsqrt(1.0 + dar[...][:, :1] + dbr[...][:, :1])
        dinvr[...] = dinv
        g1r[...] = jnp.concatenate(
            [xr[...], jnp.zeros((BN, 5), f32)], axis=1) * dinv

    return pl.pallas_call(
        body,
        grid=(NB,),
        in_specs=[_row(3), _row(8), _row(8)],
        out_specs=[_row(1), _row(8)],
        out_shape=[jax.ShapeDtypeStruct((N, 1), f32),
                   jax.ShapeDtypeStruct((N, 8), f32)],
    )(x, dega, degb)


def _tc2(s1a, s1b, g1, dinv, W1p, b1):
    def body(s1ar, s1br, g1r, dr, wr, br, g2r):
        h = pl.program_id(0)
        d = dr[...]
        p1 = d * (s1ar[...] + s1br[...] + g1r[...])
        h1 = jnp.maximum(
            jnp.dot(p1, wr[...], preferred_element_type=f32) + br[...], 0.0)
        g2 = h1 * d
        g2r[...] = jnp.where(h == 0, g2[:, :16], g2[:, 16:])

    return pl.pallas_call(
        body,
        grid=(2, NB),
        in_specs=[pl.BlockSpec((BN, 8), lambda h, i: (i, 0)),
                  pl.BlockSpec((BN, 8), lambda h, i: (i, 0)),
                  pl.BlockSpec((BN, 8), lambda h, i: (i, 0)),
                  pl.BlockSpec((BN, 1), lambda h, i: (i, 0)),
                  pl.BlockSpec((8, 32), lambda h, i: (0, 0)),
                  pl.BlockSpec((1, 32), lambda h, i: (0, 0))],
        out_specs=pl.BlockSpec((BN, 16), lambda h, i: (h * NB + i, 0)),
        out_shape=jax.ShapeDtypeStruct((2 * N, 16), f32),
    )(s1a, s1b, g1, dinv, W1p, b1)


def _tc3(s2a, s2b, g2, dinv, W2, b2, W3):
    def body(s2ar, s2br, g2ar, g2br, dr, w2r, b2r, w3r, g3r):
        d = dr[...]
        p2 = jnp.concatenate(
            [d * (s2ar[...] + g2ar[...]), d * (s2br[...] + g2br[...])], axis=1)
        h2 = jnp.maximum(
            jnp.dot(p2, w2r[...], preferred_element_type=f32) + b2r[...], 0.0)
        t3 = jnp.dot(h2, w3r[...], preferred_element_type=f32) * d
        g3r[...] = jnp.concatenate([t3, jnp.zeros((BN, 6), f32)], axis=1)

    return pl.pallas_call(
        body,
        grid=(NB,),
        in_specs=[_row(16), _row(16),
                  pl.BlockSpec((BN, 16), lambda i: (i, 0)),
                  pl.BlockSpec((BN, 16), lambda i: (NB + i, 0)),
                  _row(1),
                  pl.BlockSpec((32, 64), lambda i: (0, 0)),
                  pl.BlockSpec((1, 64), lambda i: (0, 0)),
                  pl.BlockSpec((64, 2), lambda i: (0, 0))],
        out_specs=_row(8),
        out_shape=jax.ShapeDtypeStruct((N, 8), f32),
    )(s2a, s2b, g2, g2, dinv, W2, b2, W3)


def _tc4(s3a, s3b, g3, dinv, batT, b3):
    def body(s3ar, s3br, g3r, dr, batr, b3r, outr, sums, cnt):
        i = pl.program_id(0)
        q = dr[...] * (s3ar[...][:, :2] + s3br[...][:, :2]
                       + g3r[...][:, :2])
        gid = lax.broadcasted_iota(jnp.int32, (G, BN), 0)
        maskT = (batr[0] == gid).astype(f32)

        @pl.when(i == 0)
        def _():
            sums[...] = jnp.zeros_like(sums)
            cnt[...] = jnp.zeros_like(cnt)

        sums[...] += jnp.dot(maskT, q, preferred_element_type=f32)
        cnt[...] += jnp.dot(maskT, jnp.ones((BN, 2), f32),
                            preferred_element_type=f32)

        @pl.when(i == NB - 1)
        def _():
            c = cnt[...]
            outr[...] = (sums[...] / jnp.maximum(c, 1.0)
                         + jnp.where(c > 0, b3r[...], 0.0))

    return pl.pallas_call(
        body,
        grid=(NB,),
        in_specs=[_row(8), _row(8), _row(8), _row(1),
                  pl.BlockSpec((1, 1, BN), lambda i: (i, 0, 0)),
                  pl.BlockSpec((1, 2), lambda i: (0, 0))],
        out_specs=pl.BlockSpec((G, 2), lambda i: (0, 0)),
        out_shape=jax.ShapeDtypeStruct((G, 2), f32),
        scratch_shapes=[pltpu.VMEM((G, 2), f32), pltpu.VMEM((G, 2), f32)],
    )(s3a, s3b, g3, dinv, batT, b3)


def kernel(x, edge_index, batch, W1, b1, W2, b2, W3, b3):
    src = edge_index[0]
    dst = edge_index[1]
    W1p = jnp.pad(W1, ((0, 5), (0, 0)))

    dega, degb = _get_sc_deg()(dst, jnp.ones((K, 8), f32),
                               jnp.zeros((N_PAD, 8), f32))
    dinv, g1 = _tc1(x, dega, degb)

    s1a, s1b = _make_sc_scatter(8, KS, True)(
        g1, src, dst, jnp.zeros((N_PAD, 8), f32))
    g2 = _tc2(s1a, s1b, g1, dinv, W1p, b1.reshape(1, 32))

    srcs = jnp.concatenate([src, src + N])
    s2a, s2b = _make_sc_scatter(16, KW, False)(
        g2, srcs, dst, jnp.zeros((N_PAD, 16), f32))
    g3 = _tc3(s2a, s2b, g2, dinv, W2, b2.reshape(1, 64), W3)

    s3a, s3b = _make_sc_scatter(8, KS, True)(
        g3, src, dst, jnp.zeros((N_PAD, 8), f32))
    return _tc4(s3a, s3b, g3, dinv,
                batch.reshape(NB, 1, BN), b3.reshape(1, 2))
